# baseline (speedup 1.0000x reference)
import functools

import jax
import jax.numpy as jnp
from jax import lax
from jax.experimental import pallas as pl
from jax.experimental.pallas import tpu as pltpu
from jax.experimental.pallas import tpu_sc as plsc

B, S, D = 4, 4096, 2048
ROWS = B * S
NC, NS = 2, 16
NW = NC * NS
LANES = 16
GROUP = 16
NV = D // LANES

SC_ROWS = 1024
TC_ROWS = ROWS - SC_ROWS
ROWS_W = SC_ROWS // NW
NGROUP = ROWS_W // GROUP

BLK = 256
CH = 256

_mesh = plsc.VectorSubcoreMesh(core_axis_name="c", subcore_axis_name="s")


@functools.partial(
    pl.kernel,
    mesh=_mesh,
    out_type=jax.ShapeDtypeStruct((ROWS, D), jnp.float32),
    scratch_types=[
        pltpu.VMEM((GROUP, D), jnp.float32),
        pltpu.VMEM((GROUP, D), jnp.float32),
        pltpu.SemaphoreType.DMA,
        pltpu.SemaphoreType.DMA,
        pltpu.SemaphoreType.DMA,
        pltpu.SemaphoreType.DMA,
    ],
    compiler_params=pltpu.CompilerParams(needs_layout_passes=False),
)
def _cumsum_rows_sc(x_hbm, out_hbm, buf0, buf1, isem0, isem1, osem0, osem1):
    wid = lax.axis_index("s") * NC + lax.axis_index("c")
    base = wid * ROWS_W
    bufs = (buf0, buf1)
    isems = (isem0, isem1)
    osems = (osem0, osem1)

    def in_copy(g, s):
        return pltpu.make_async_copy(
            x_hbm.at[pl.ds(base + g * GROUP, GROUP)], bufs[s], isems[s])

    def out_copy(g, s):
        return pltpu.make_async_copy(
            bufs[s], out_hbm.at[pl.ds(base + g * GROUP, GROUP)], osems[s])

    lane0 = lax.iota(jnp.int32, LANES) == 0

    def compute(buf):
        def step(i, carries):
            off = i * LANES
            new = []
            for r in range(GROUP):
                v = buf[r, pl.ds(off, LANES)]
                s = plsc.cumsum(v)
                total = plsc.cumsum(lax.rev(s, (0,)), mask=lane0)
                buf[r, pl.ds(off, LANES)] = s + carries[r]
                new.append(carries[r] + total)
            return tuple(new)

        lax.fori_loop(
            0, NV, step,
            tuple(jnp.zeros((LANES,), jnp.float32) for _ in range(GROUP)))

    in_copy(0, 0).start()
    in_copy(1, 1).start()

    def gbody(gg, carry):
        for s in range(2):
            g = gg * 2 + s

            in_copy(g, s).wait()

            @pl.when(gg > 0)
            def _():
                out_copy(g - 2, s).wait()

            compute(bufs[s])
            out_copy(g, s).start()

            @pl.when(g + 2 < NGROUP)
            def _():
                in_copy(g + 2, s).start()
        return carry

    lax.fori_loop(0, NGROUP // 2, gbody, 0)
    out_copy(NGROUP - 2, 0).wait()
    out_copy(NGROUP - 1, 1).wait()


def _tc_body(x_ref, tri_ref, scfull_ref, o_ref):
    del scfull_ref
    tri = tri_ref[...]
    carry = jnp.zeros((BLK, 1), jnp.float32)
    for c in range(D // CH):
        xc = x_ref[:, c * CH:(c + 1) * CH]
        sc = lax.dot_general(xc, tri, (((1,), (0,)), ((), ())),
                             preferred_element_type=jnp.float32)
        oc = sc + carry
        o_ref[:, c * CH:(c + 1) * CH] = oc
        carry = oc[:, CH - 1:CH]


_cumsum_rows_tc = pl.pallas_call(
    _tc_body,
    grid=(TC_ROWS // BLK,),
    in_specs=[
        pl.BlockSpec((BLK, D), lambda i: (i + SC_ROWS // BLK, 0)),
        pl.BlockSpec((CH, CH), lambda i: (0, 0)),
        pl.BlockSpec(memory_space=pl.ANY),
    ],
    out_specs=pl.BlockSpec((BLK, D), lambda i: (i + SC_ROWS // BLK, 0)),
    out_shape=jax.ShapeDtypeStruct((ROWS, D), jnp.float32),
    input_output_aliases={2: 0},
)


def kernel(x):
    xf = x.reshape(ROWS, D)
    tri = jnp.triu(jnp.ones((CH, CH), jnp.float32))
    sc_full = _cumsum_rows_sc(xf)
    out = _cumsum_rows_tc(xf, tri, sc_full)
    return out.reshape(B, S, D)


# speedup vs baseline: 1.1026x; 1.1026x over previous
import functools

import jax
import jax.numpy as jnp
from jax import lax
from jax.experimental import pallas as pl
from jax.experimental.pallas import tpu as pltpu
from jax.experimental.pallas import tpu_sc as plsc

B, S, D = 4, 4096, 2048
ROWS = B * S
NC, NS = 2, 16
NW = NC * NS
LANES = 16
GROUP = 16
NV = D // LANES

SC_ROWS = 1024
TC_ROWS = ROWS - SC_ROWS
ROWS_W = SC_ROWS // NW
NGROUP = ROWS_W // GROUP

BLK = 256
CH = 256

_mesh = plsc.VectorSubcoreMesh(core_axis_name="c", subcore_axis_name="s")


@functools.partial(
    pl.kernel,
    mesh=_mesh,
    out_type=jax.ShapeDtypeStruct((SC_ROWS, D), jnp.float32),
    scratch_types=[
        pltpu.VMEM((GROUP, D), jnp.float32),
        pltpu.VMEM((GROUP, D), jnp.float32),
        pltpu.SemaphoreType.DMA,
        pltpu.SemaphoreType.DMA,
        pltpu.SemaphoreType.DMA,
        pltpu.SemaphoreType.DMA,
    ],
    compiler_params=pltpu.CompilerParams(needs_layout_passes=False),
)
def _cumsum_rows_sc(x_hbm, out_hbm, buf0, buf1, isem0, isem1, osem0, osem1):
    wid = lax.axis_index("s") * NC + lax.axis_index("c")
    base = wid * ROWS_W
    bufs = (buf0, buf1)
    isems = (isem0, isem1)
    osems = (osem0, osem1)

    def in_copy(g, s):
        return pltpu.make_async_copy(
            x_hbm.at[pl.ds(base + g * GROUP, GROUP)], bufs[s], isems[s])

    def out_copy(g, s):
        return pltpu.make_async_copy(
            bufs[s], out_hbm.at[pl.ds(base + g * GROUP, GROUP)], osems[s])

    lane0 = lax.iota(jnp.int32, LANES) == 0

    def compute(buf):
        def step(i, carries):
            off = i * LANES
            new = []
            for r in range(GROUP):
                v = buf[r, pl.ds(off, LANES)]
                s = plsc.cumsum(v)
                total = plsc.cumsum(lax.rev(s, (0,)), mask=lane0)
                buf[r, pl.ds(off, LANES)] = s + carries[r]
                new.append(carries[r] + total)
            return tuple(new)

        lax.fori_loop(
            0, NV, step,
            tuple(jnp.zeros((LANES,), jnp.float32) for _ in range(GROUP)))

    in_copy(0, 0).start()
    in_copy(1, 1).start()

    def gbody(gg, carry):
        for s in range(2):
            g = gg * 2 + s

            in_copy(g, s).wait()

            @pl.when(gg > 0)
            def _():
                out_copy(g - 2, s).wait()

            compute(bufs[s])
            out_copy(g, s).start()

            @pl.when(g + 2 < NGROUP)
            def _():
                in_copy(g + 2, s).start()
        return carry

    lax.fori_loop(0, NGROUP // 2, gbody, 0)
    out_copy(NGROUP - 2, 0).wait()
    out_copy(NGROUP - 1, 1).wait()


def _tc_body(x_ref, tri_ref, o_ref):
    tri = tri_ref[...]
    carry = jnp.zeros((BLK, 1), jnp.float32)
    for c in range(D // CH):
        xc = x_ref[:, c * CH:(c + 1) * CH]
        sc = lax.dot_general(xc, tri, (((1,), (0,)), ((), ())),
                             preferred_element_type=jnp.float32)
        oc = sc + carry
        o_ref[:, c * CH:(c + 1) * CH] = oc
        carry = oc[:, CH - 1:CH]


_cumsum_rows_tc = pl.pallas_call(
    _tc_body,
    grid=(TC_ROWS // BLK,),
    in_specs=[
        pl.BlockSpec((BLK, D), lambda i: (i + SC_ROWS // BLK, 0)),
        pl.BlockSpec((CH, CH), lambda i: (0, 0)),
    ],
    out_specs=pl.BlockSpec((BLK, D), lambda i: (i + SC_ROWS // BLK, 0)),
    out_shape=jax.ShapeDtypeStruct((ROWS, D), jnp.float32),
)


def kernel(x):
    xf = x.reshape(ROWS, D)
    tri = jnp.triu(jnp.ones((CH, CH), jnp.float32))
    sc_out = _cumsum_rows_sc(xf)
    tc_full = _cumsum_rows_tc(xf, tri)
    out = lax.dynamic_update_slice(tc_full, sc_out, (0, 0))
    return out.reshape(B, S, D)
